# baseline (speedup 1.0000x reference)
import functools

import jax
import jax.numpy as jnp
from jax import lax
from jax.experimental import pallas as pl
from jax.experimental.pallas import tpu as pltpu
from jax.experimental.pallas import tpu_sc as plsc

N_NODES = 10000
N_EDGES = 320000
HIDDEN = 128
MIDDLE = 320
DICT = 512

NUM_CORES = 2
NUM_SUBCORES = 16
NUM_WORKERS = NUM_CORES * NUM_SUBCORES
BATCH = 80
BPC = 25
CHUNKS = N_EDGES // (NUM_WORKERS * BPC * BATCH)
STRIPE = 624
TAIL = N_NODES - NUM_SUBCORES * STRIPE


@functools.partial(
    pl.kernel,
    mesh=plsc.VectorSubcoreMesh(core_axis_name="c", subcore_axis_name="s"),
    out_type=jax.ShapeDtypeStruct((NUM_CORES, N_NODES, HIDDEN), jnp.float32),
    scratch_types=[
        pltpu.VMEM((BPC, BATCH), jnp.int32),
        pltpu.VMEM((BPC, BATCH), jnp.int32),
        pltpu.VMEM((BATCH, HIDDEN), jnp.float32),
        pltpu.VMEM((BATCH, HIDDEN), jnp.float32),
        pltpu.VMEM_SHARED((N_NODES, HIDDEN), jnp.float32),
        pltpu.SemaphoreType.DMA,
        pltpu.SemaphoreType.DMA,
        pltpu.SemaphoreType.DMA,
        pltpu.SemaphoreType.DMA,
    ],
)
def _sc_agg(x_hbm, src_hbm, dst_hbm, zeros_hbm, out_hbm, src_v, dst_v,
            r0, r1, acc, g0, g1, s0, s1):
    c = lax.axis_index("c")
    s = lax.axis_index("s")
    wid = s * NUM_CORES + c

    pltpu.sync_copy(zeros_hbm, acc.at[pl.ds(s * STRIPE, STRIPE)])

    @pl.when(s == 0)
    def _zero_tail():
        pltpu.sync_copy(
            zeros_hbm.at[pl.ds(0, TAIL)],
            acc.at[pl.ds(NUM_SUBCORES * STRIPE, TAIL)],
        )

    plsc.subcore_barrier()

    rows = (r0, r1)
    gsem = (g0, g1)
    ssem = (s0, s1)

    def _wait_gather(u):
        pltpu.make_async_copy(x_hbm.at[src_v.at[0]], rows[u], gsem[u]).wait()

    def _wait_scatter(u):
        pltpu.make_async_copy(rows[u], acc.at[dst_v.at[0]], ssem[u]).wait()

    def _chunk(k, carry):
        pltpu.sync_copy(src_hbm.at[wid, k], src_v)
        pltpu.sync_copy(dst_hbm.at[wid, k], dst_v)
        pltpu.async_copy(x_hbm.at[src_v.at[0]], rows[0], gsem[0])
        _wait_gather(0)
        pltpu.async_copy(rows[0], acc.at[dst_v.at[0]], ssem[0], add=True)
        pltpu.async_copy(x_hbm.at[src_v.at[1]], rows[1], gsem[1])

        def _pair(i, cc):
            j = 2 * i + 1
            _wait_gather(1)
            pltpu.async_copy(rows[1], acc.at[dst_v.at[j]], ssem[1], add=True)
            _wait_scatter(0)
            pltpu.async_copy(x_hbm.at[src_v.at[j + 1]], rows[0], gsem[0])
            _wait_gather(0)
            pltpu.async_copy(rows[0], acc.at[dst_v.at[j + 1]], ssem[0],
                             add=True)

            @pl.when(i < (BPC - 3) // 2)
            def _():
                _wait_scatter(1)
                pltpu.async_copy(x_hbm.at[src_v.at[j + 2]], rows[1], gsem[1])

            return cc

        lax.fori_loop(0, (BPC - 1) // 2, _pair, 0)
        _wait_scatter(0)
        _wait_scatter(1)
        return carry

    lax.fori_loop(0, CHUNKS, _chunk, 0)
    plsc.subcore_barrier()

    pltpu.sync_copy(
        acc.at[pl.ds(s * STRIPE, STRIPE)],
        out_hbm.at[c, pl.ds(s * STRIPE, STRIPE)],
    )

    @pl.when(s == 0)
    def _write_tail():
        pltpu.sync_copy(
            acc.at[pl.ds(NUM_SUBCORES * STRIPE, TAIL)],
            out_hbm.at[c, pl.ds(NUM_SUBCORES * STRIPE, TAIL)],
        )


def _mlp_body(x_ref, a0_ref, a1_ref, tw_ref, tb_ref, pw_ref, pb_ref, o_ref):
    h = x_ref[...] + a0_ref[...] + a1_ref[...]
    h1 = jnp.dot(h, tw_ref[...], preferred_element_type=jnp.float32) + tb_ref[...]
    o_ref[...] = jnp.dot(h1, pw_ref[...], preferred_element_type=jnp.float32) + pb_ref[...]


ROW_BLOCK = 1000


def _tc_mlp(x, a0, a1, trn_w, trn_b, prd_w, prd_b):
    return pl.pallas_call(
        _mlp_body,
        grid=(N_NODES // ROW_BLOCK,),
        in_specs=[
            pl.BlockSpec((ROW_BLOCK, HIDDEN), lambda i: (i, 0)),
            pl.BlockSpec((ROW_BLOCK, HIDDEN), lambda i: (i, 0)),
            pl.BlockSpec((ROW_BLOCK, HIDDEN), lambda i: (i, 0)),
            pl.BlockSpec((HIDDEN, MIDDLE), lambda i: (0, 0)),
            pl.BlockSpec((1, MIDDLE), lambda i: (0, 0)),
            pl.BlockSpec((MIDDLE, DICT), lambda i: (0, 0)),
            pl.BlockSpec((1, DICT), lambda i: (0, 0)),
        ],
        out_specs=pl.BlockSpec((ROW_BLOCK, DICT), lambda i: (i, 0)),
        out_shape=jax.ShapeDtypeStruct((N_NODES, DICT), jnp.float32),
    )(x, a0, a1, trn_w, trn_b, prd_w, prd_b)


def kernel(x, edge_index, trn_w, trn_b, prd_w, prd_b):
    ei = edge_index.astype(jnp.int32)
    src = ei[0].reshape(NUM_WORKERS, CHUNKS, BPC, BATCH)
    dst = ei[1].reshape(NUM_WORKERS, CHUNKS, BPC, BATCH)
    zeros = jnp.zeros((STRIPE, HIDDEN), jnp.float32)
    agg = _sc_agg(x, src, dst, zeros)
    return _tc_mlp(
        x,
        agg[0],
        agg[1],
        trn_w,
        trn_b.reshape(1, MIDDLE),
        prd_w,
        prd_b.reshape(1, DICT),
    )


# speedup vs baseline: 1.2698x; 1.2698x over previous
import functools

import jax
import jax.numpy as jnp
from jax import lax
from jax.experimental import pallas as pl
from jax.experimental.pallas import tpu as pltpu
from jax.experimental.pallas import tpu_sc as plsc

N_NODES = 10000
N_EDGES = 320000
HIDDEN = 128
MIDDLE = 320
DICT = 512

NUM_CORES = 2
NUM_SUBCORES = 16
NUM_WORKERS = NUM_CORES * NUM_SUBCORES
BATCH = 40
BPC = 25
CHUNKS = N_EDGES // (NUM_WORKERS * BPC * BATCH)
STRIPE = 624
TAIL = N_NODES - NUM_SUBCORES * STRIPE


@functools.partial(
    pl.kernel,
    mesh=plsc.VectorSubcoreMesh(core_axis_name="c", subcore_axis_name="s"),
    out_type=jax.ShapeDtypeStruct((NUM_CORES, N_NODES, HIDDEN), jnp.float32),
    scratch_types=[
        pltpu.VMEM((BPC, BATCH), jnp.int32),
        pltpu.VMEM((BPC, BATCH), jnp.int32),
        pltpu.VMEM((BATCH, HIDDEN), jnp.float32),
        pltpu.VMEM((BATCH, HIDDEN), jnp.float32),
        pltpu.VMEM((BATCH, HIDDEN), jnp.float32),
        pltpu.VMEM((BATCH, HIDDEN), jnp.float32),
        pltpu.VMEM_SHARED((N_NODES, HIDDEN), jnp.float32),
        pltpu.SemaphoreType.DMA,
        pltpu.SemaphoreType.DMA,
        pltpu.SemaphoreType.DMA,
        pltpu.SemaphoreType.DMA,
        pltpu.SemaphoreType.DMA,
        pltpu.SemaphoreType.DMA,
        pltpu.SemaphoreType.DMA,
        pltpu.SemaphoreType.DMA,
    ],
)
def _sc_agg(x_hbm, ei_hbm, zeros_hbm, out_hbm, src_v, dst_v,
            r0, r1, r2, r3, acc, g0, g1, g2, g3, s0, s1, s2, s3):
    c = lax.axis_index("c")
    s = lax.axis_index("s")
    wid = s * NUM_CORES + c

    pltpu.sync_copy(zeros_hbm, acc.at[pl.ds(s * STRIPE, STRIPE)])

    @pl.when(s == 0)
    def _zero_tail():
        pltpu.sync_copy(
            zeros_hbm.at[pl.ds(0, TAIL)],
            acc.at[pl.ds(NUM_SUBCORES * STRIPE, TAIL)],
        )

    plsc.subcore_barrier()

    rows = (r0, r1, r2, r3)
    gsem = (g0, g1, g2, g3)
    ssem = (s0, s1, s2, s3)

    def _wait_gather(u):
        pltpu.make_async_copy(x_hbm.at[src_v.at[0]], rows[u], gsem[u]).wait()

    def _wait_scatter(u):
        pltpu.make_async_copy(rows[u], acc.at[dst_v.at[0]], ssem[u]).wait()

    def _step(j, issue_j):
        u = j % 4
        _wait_gather(u)
        pltpu.async_copy(rows[u], acc.at[dst_v.at[j]], ssem[u], add=True)
        if issue_j is not None:
            v = issue_j % 4
            if issue_j >= 4:
                _wait_scatter(v)
            pltpu.async_copy(x_hbm.at[src_v.at[issue_j]], rows[v], gsem[v])

    def _chunk(k, carry):
        pltpu.sync_copy(ei_hbm.at[0, wid, k], src_v)
        pltpu.sync_copy(ei_hbm.at[1, wid, k], dst_v)
        for j in range(3):
            pltpu.async_copy(x_hbm.at[src_v.at[j]], rows[j], gsem[j])

        def _quad(i, cc):
            j0 = 4 * i

            def _dyn_step(u):
                _wait_gather(u)
                pltpu.async_copy(rows[u], acc.at[dst_v.at[j0 + u]], ssem[u],
                                 add=True)
                v = (u + 3) % 4
                @pl.when(j0 + u >= 1)
                def _():
                    _wait_scatter(v)
                pltpu.async_copy(x_hbm.at[src_v.at[j0 + u + 3]], rows[v],
                                 gsem[v])

            for u in range(4):
                _dyn_step(u)
            return cc

        lax.fori_loop(0, 5, _quad, 0)
        _step(20, 23)
        _step(21, 24)
        _step(22, None)
        _step(23, None)
        _step(24, None)
        for u in range(4):
            _wait_scatter(u)
        return carry

    lax.fori_loop(0, CHUNKS, _chunk, 0)
    plsc.subcore_barrier()

    pltpu.sync_copy(
        acc.at[pl.ds(s * STRIPE, STRIPE)],
        out_hbm.at[c, pl.ds(s * STRIPE, STRIPE)],
    )

    @pl.when(s == 0)
    def _write_tail():
        pltpu.sync_copy(
            acc.at[pl.ds(NUM_SUBCORES * STRIPE, TAIL)],
            out_hbm.at[c, pl.ds(NUM_SUBCORES * STRIPE, TAIL)],
        )


def _mlp_body(x_ref, a_ref, tw_ref, tb_ref, pw_ref, pb_ref, o_ref):
    h = x_ref[...] + a_ref[0] + a_ref[1]
    h1 = jnp.dot(h, tw_ref[...], preferred_element_type=jnp.float32) + tb_ref[...]
    o_ref[...] = jnp.dot(h1, pw_ref[...], preferred_element_type=jnp.float32) + pb_ref[...]


ROW_BLOCK = 1000


def _tc_mlp(x, agg, trn_w, trn_b, prd_w, prd_b):
    return pl.pallas_call(
        _mlp_body,
        grid=(N_NODES // ROW_BLOCK,),
        in_specs=[
            pl.BlockSpec((ROW_BLOCK, HIDDEN), lambda i: (i, 0)),
            pl.BlockSpec((NUM_CORES, ROW_BLOCK, HIDDEN), lambda i: (0, i, 0)),
            pl.BlockSpec((HIDDEN, MIDDLE), lambda i: (0, 0)),
            pl.BlockSpec((1, MIDDLE), lambda i: (0, 0)),
            pl.BlockSpec((MIDDLE, DICT), lambda i: (0, 0)),
            pl.BlockSpec((1, DICT), lambda i: (0, 0)),
        ],
        out_specs=pl.BlockSpec((ROW_BLOCK, DICT), lambda i: (i, 0)),
        out_shape=jax.ShapeDtypeStruct((N_NODES, DICT), jnp.float32),
    )(x, agg, trn_w, trn_b, prd_w, prd_b)


def kernel(x, edge_index, trn_w, trn_b, prd_w, prd_b):
    ei = edge_index.astype(jnp.int32).reshape(2, NUM_WORKERS, CHUNKS, BPC, BATCH)
    zeros = jnp.zeros((STRIPE, HIDDEN), jnp.float32)
    agg = _sc_agg(x, ei, zeros)
    return _tc_mlp(
        x,
        agg,
        trn_w,
        trn_b.reshape(1, MIDDLE),
        prd_w,
        prd_b.reshape(1, DICT),
    )


# speedup vs baseline: 1.2886x; 1.0149x over previous
import functools

import jax
import jax.numpy as jnp
from jax import lax
from jax.experimental import pallas as pl
from jax.experimental.pallas import tpu as pltpu
from jax.experimental.pallas import tpu_sc as plsc

N_NODES = 10000
N_EDGES = 320000
HIDDEN = 128
MIDDLE = 320
DICT = 512

NUM_CORES = 2
NUM_SUBCORES = 16
NUM_WORKERS = NUM_CORES * NUM_SUBCORES
BATCH = 40
BPC = 25
CHUNKS = N_EDGES // (NUM_WORKERS * BPC * BATCH)
STRIPE = 624
TAIL = N_NODES - NUM_SUBCORES * STRIPE


@functools.partial(
    pl.kernel,
    mesh=plsc.VectorSubcoreMesh(core_axis_name="c", subcore_axis_name="s"),
    out_type=jax.ShapeDtypeStruct((NUM_CORES, N_NODES, HIDDEN), jnp.float32),
    scratch_types=[
        pltpu.VMEM((BPC, BATCH), jnp.int32),
        pltpu.VMEM((BPC, BATCH), jnp.int32),
        pltpu.VMEM((BATCH, HIDDEN), jnp.float32),
        pltpu.VMEM((BATCH, HIDDEN), jnp.float32),
        pltpu.VMEM((BATCH, HIDDEN), jnp.float32),
        pltpu.VMEM((BATCH, HIDDEN), jnp.float32),
        pltpu.VMEM_SHARED((N_NODES, HIDDEN), jnp.float32),
        pltpu.SemaphoreType.DMA,
        pltpu.SemaphoreType.DMA,
        pltpu.SemaphoreType.DMA,
        pltpu.SemaphoreType.DMA,
        pltpu.SemaphoreType.DMA,
        pltpu.SemaphoreType.DMA,
        pltpu.SemaphoreType.DMA,
        pltpu.SemaphoreType.DMA,
    ],
)
def _sc_agg(x_hbm, ei_hbm, zeros_hbm, out_hbm, src_v, dst_v,
            r0, r1, r2, r3, acc, g0, g1, g2, g3, s0, s1, s2, s3):
    c = lax.axis_index("c")
    s = lax.axis_index("s")
    wid = s * NUM_CORES + c

    pltpu.sync_copy(zeros_hbm, acc.at[pl.ds(s * STRIPE, STRIPE)])

    @pl.when(s == 0)
    def _zero_tail():
        pltpu.sync_copy(
            zeros_hbm.at[pl.ds(0, TAIL)],
            acc.at[pl.ds(NUM_SUBCORES * STRIPE, TAIL)],
        )

    plsc.subcore_barrier()

    rows = (r0, r1, r2, r3)
    gsem = (g0, g1, g2, g3)
    ssem = (s0, s1, s2, s3)

    def _wait_gather(u):
        pltpu.make_async_copy(x_hbm.at[src_v.at[0]], rows[u], gsem[u]).wait()

    def _wait_scatter(u):
        pltpu.make_async_copy(rows[u], acc.at[dst_v.at[0]], ssem[u]).wait()

    def _step(j, issue_j):
        u = j % 4
        _wait_gather(u)
        pltpu.async_copy(rows[u], acc.at[dst_v.at[j]], ssem[u], add=True)
        if issue_j is not None:
            v = issue_j % 4
            if issue_j >= 4:
                _wait_scatter(v)
            pltpu.async_copy(x_hbm.at[src_v.at[issue_j]], rows[v], gsem[v])

    def _chunk(k, carry):
        pltpu.sync_copy(ei_hbm.at[0, wid, k], src_v)
        pltpu.sync_copy(ei_hbm.at[1, wid, k], dst_v)
        for j in range(3):
            pltpu.async_copy(x_hbm.at[src_v.at[j]], rows[j], gsem[j])

        def _quad(i, cc):
            j0 = 4 * i

            def _dyn_step(u):
                _wait_gather(u)
                pltpu.async_copy(rows[u], acc.at[dst_v.at[j0 + u]], ssem[u],
                                 add=True)
                v = (u + 3) % 4
                @pl.when(j0 + u >= 1)
                def _():
                    _wait_scatter(v)
                pltpu.async_copy(x_hbm.at[src_v.at[j0 + u + 3]], rows[v],
                                 gsem[v])

            for u in range(4):
                _dyn_step(u)
            return cc

        lax.fori_loop(0, 5, _quad, 0)
        _step(20, 23)
        _step(21, 24)
        _step(22, None)
        _step(23, None)
        _step(24, None)
        for u in range(4):
            _wait_scatter(u)
        return carry

    lax.fori_loop(0, CHUNKS, _chunk, 0)
    plsc.subcore_barrier()

    pltpu.sync_copy(
        acc.at[pl.ds(s * STRIPE, STRIPE)],
        out_hbm.at[c, pl.ds(s * STRIPE, STRIPE)],
    )

    @pl.when(s == 0)
    def _write_tail():
        pltpu.sync_copy(
            acc.at[pl.ds(NUM_SUBCORES * STRIPE, TAIL)],
            out_hbm.at[c, pl.ds(NUM_SUBCORES * STRIPE, TAIL)],
        )


def _fuse_body(tw_ref, tb_ref, pw_ref, pb_ref, w_ref, b_ref):
    w_ref[...] = jnp.dot(tw_ref[...], pw_ref[...],
                         preferred_element_type=jnp.float32)
    b_ref[...] = jnp.dot(tb_ref[...], pw_ref[...],
                         preferred_element_type=jnp.float32) + pb_ref[...]


def _fuse_weights(trn_w, trn_b, prd_w, prd_b):
    return pl.pallas_call(
        _fuse_body,
        out_shape=(
            jax.ShapeDtypeStruct((HIDDEN, DICT), jnp.float32),
            jax.ShapeDtypeStruct((1, DICT), jnp.float32),
        ),
    )(trn_w, trn_b, prd_w, prd_b)


def _mlp_body(x_ref, a_ref, w_ref, b_ref, o_ref):
    h = x_ref[...] + a_ref[0] + a_ref[1]
    o_ref[...] = jnp.dot(h, w_ref[...],
                         preferred_element_type=jnp.float32) + b_ref[...]


ROW_BLOCK = 1000


def _tc_mlp(x, agg, w, b):
    return pl.pallas_call(
        _mlp_body,
        grid=(N_NODES // ROW_BLOCK,),
        in_specs=[
            pl.BlockSpec((ROW_BLOCK, HIDDEN), lambda i: (i, 0)),
            pl.BlockSpec((NUM_CORES, ROW_BLOCK, HIDDEN), lambda i: (0, i, 0)),
            pl.BlockSpec((HIDDEN, DICT), lambda i: (0, 0)),
            pl.BlockSpec((1, DICT), lambda i: (0, 0)),
        ],
        out_specs=pl.BlockSpec((ROW_BLOCK, DICT), lambda i: (i, 0)),
        out_shape=jax.ShapeDtypeStruct((N_NODES, DICT), jnp.float32),
    )(x, agg, w, b)


def kernel(x, edge_index, trn_w, trn_b, prd_w, prd_b):
    ei = edge_index.astype(jnp.int32).reshape(2, NUM_WORKERS, CHUNKS, BPC, BATCH)
    zeros = jnp.zeros((STRIPE, HIDDEN), jnp.float32)
    w, b = _fuse_weights(trn_w, trn_b.reshape(1, MIDDLE), prd_w,
                         prd_b.reshape(1, DICT))
    agg = _sc_agg(x, ei, zeros)
    return _tc_mlp(x, agg, w, b)


# speedup vs baseline: 1.4491x; 1.1245x over previous
import functools

import jax
import jax.numpy as jnp
from jax import lax
from jax.experimental import pallas as pl
from jax.experimental.pallas import tpu as pltpu
from jax.experimental.pallas import tpu_sc as plsc

N_NODES = 10000
N_EDGES = 320000
HIDDEN = 128
MIDDLE = 320
DICT = 512

NUM_CORES = 2
NUM_SUBCORES = 16
NUM_WORKERS = NUM_CORES * NUM_SUBCORES
BATCH = 80
BPC = 25
CHUNKS = N_EDGES // (NUM_WORKERS * BPC * BATCH)
STRIPE = 624
TAIL = N_NODES - NUM_SUBCORES * STRIPE


@functools.partial(
    pl.kernel,
    mesh=plsc.VectorSubcoreMesh(core_axis_name="c", subcore_axis_name="s"),
    out_type=jax.ShapeDtypeStruct((NUM_CORES, N_NODES, HIDDEN), jnp.float32),
    scratch_types=[
        pltpu.VMEM((BPC, BATCH), jnp.int32),
        pltpu.VMEM((BPC, BATCH), jnp.int32),
        pltpu.VMEM((BATCH, HIDDEN), jnp.float32),
        pltpu.VMEM((BATCH, HIDDEN), jnp.float32),
        pltpu.VMEM((BATCH, HIDDEN), jnp.float32),
        pltpu.VMEM((BATCH, HIDDEN), jnp.float32),
        pltpu.VMEM_SHARED((N_NODES, HIDDEN), jnp.float32),
        pltpu.SemaphoreType.DMA,
        pltpu.SemaphoreType.DMA,
        pltpu.SemaphoreType.DMA,
        pltpu.SemaphoreType.DMA,
        pltpu.SemaphoreType.DMA,
        pltpu.SemaphoreType.DMA,
        pltpu.SemaphoreType.DMA,
        pltpu.SemaphoreType.DMA,
    ],
)
def _sc_agg(x_hbm, ei_hbm, zeros_hbm, out_hbm, src_v, dst_v,
            r0, r1, r2, r3, acc, g0, g1, g2, g3, s0, s1, s2, s3):
    c = lax.axis_index("c")
    s = lax.axis_index("s")
    wid = s * NUM_CORES + c

    pltpu.sync_copy(zeros_hbm, acc.at[pl.ds(s * STRIPE, STRIPE)])

    @pl.when(s == 0)
    def _zero_tail():
        pltpu.sync_copy(
            zeros_hbm.at[pl.ds(0, TAIL)],
            acc.at[pl.ds(NUM_SUBCORES * STRIPE, TAIL)],
        )

    plsc.subcore_barrier()

    rows = (r0, r1, r2, r3)
    gsem = (g0, g1, g2, g3)
    ssem = (s0, s1, s2, s3)

    def _wait_gather(u):
        pltpu.make_async_copy(x_hbm.at[src_v.at[0]], rows[u], gsem[u]).wait()

    def _wait_scatter(u):
        pltpu.make_async_copy(rows[u], acc.at[dst_v.at[0]], ssem[u]).wait()

    def _step(j, issue_j):
        u = j % 4
        _wait_gather(u)
        pltpu.async_copy(rows[u], acc.at[dst_v.at[j]], ssem[u], add=True)
        if issue_j is not None:
            v = issue_j % 4
            if issue_j >= 4:
                _wait_scatter(v)
            pltpu.async_copy(x_hbm.at[src_v.at[issue_j]], rows[v], gsem[v])

    def _chunk(k, carry):
        pltpu.sync_copy(ei_hbm.at[0, wid, k], src_v)
        pltpu.sync_copy(ei_hbm.at[1, wid, k], dst_v)
        for j in range(3):
            pltpu.async_copy(x_hbm.at[src_v.at[j]], rows[j], gsem[j])

        def _quad(i, cc):
            j0 = 4 * i

            def _dyn_step(u):
                _wait_gather(u)
                pltpu.async_copy(rows[u], acc.at[dst_v.at[j0 + u]], ssem[u],
                                 add=True)
                v = (u + 3) % 4
                @pl.when(j0 + u >= 1)
                def _():
                    _wait_scatter(v)
                pltpu.async_copy(x_hbm.at[src_v.at[j0 + u + 3]], rows[v],
                                 gsem[v])

            for u in range(4):
                _dyn_step(u)
            return cc

        lax.fori_loop(0, (BPC - 5) // 4, _quad, 0)
        _step(BPC - 5, BPC - 2)
        _step(BPC - 4, BPC - 1)
        _step(BPC - 3, None)
        _step(BPC - 2, None)
        _step(BPC - 1, None)
        for u in range(4):
            _wait_scatter(u)
        return carry

    lax.fori_loop(0, CHUNKS, _chunk, 0)
    plsc.subcore_barrier()

    pltpu.sync_copy(
        acc.at[pl.ds(s * STRIPE, STRIPE)],
        out_hbm.at[c, pl.ds(s * STRIPE, STRIPE)],
    )

    @pl.when(s == 0)
    def _write_tail():
        pltpu.sync_copy(
            acc.at[pl.ds(NUM_SUBCORES * STRIPE, TAIL)],
            out_hbm.at[c, pl.ds(NUM_SUBCORES * STRIPE, TAIL)],
        )


def _fuse_body(tw_ref, tb_ref, pw_ref, pb_ref, w_ref, b_ref):
    w_ref[...] = jnp.dot(tw_ref[...], pw_ref[...],
                         preferred_element_type=jnp.float32)
    b_ref[...] = jnp.dot(tb_ref[...], pw_ref[...],
                         preferred_element_type=jnp.float32) + pb_ref[...]


def _fuse_weights(trn_w, trn_b, prd_w, prd_b):
    return pl.pallas_call(
        _fuse_body,
        out_shape=(
            jax.ShapeDtypeStruct((HIDDEN, DICT), jnp.float32),
            jax.ShapeDtypeStruct((1, DICT), jnp.float32),
        ),
    )(trn_w, trn_b, prd_w, prd_b)


def _mlp_body(x_ref, a_ref, w_ref, b_ref, o_ref):
    h = x_ref[...] + a_ref[0] + a_ref[1]
    o_ref[...] = jnp.dot(h, w_ref[...],
                         preferred_element_type=jnp.float32) + b_ref[...]


ROW_BLOCK = 1000


def _tc_mlp(x, agg, w, b):
    return pl.pallas_call(
        _mlp_body,
        grid=(N_NODES // ROW_BLOCK,),
        in_specs=[
            pl.BlockSpec((ROW_BLOCK, HIDDEN), lambda i: (i, 0)),
            pl.BlockSpec((NUM_CORES, ROW_BLOCK, HIDDEN), lambda i: (0, i, 0)),
            pl.BlockSpec((HIDDEN, DICT), lambda i: (0, 0)),
            pl.BlockSpec((1, DICT), lambda i: (0, 0)),
        ],
        out_specs=pl.BlockSpec((ROW_BLOCK, DICT), lambda i: (i, 0)),
        out_shape=jax.ShapeDtypeStruct((N_NODES, DICT), jnp.float32),
    )(x, agg, w, b)


def kernel(x, edge_index, trn_w, trn_b, prd_w, prd_b):
    ei = edge_index.astype(jnp.int32).reshape(2, NUM_WORKERS, CHUNKS, BPC, BATCH)
    zeros = jnp.zeros((STRIPE, HIDDEN), jnp.float32)
    w, b = _fuse_weights(trn_w, trn_b.reshape(1, MIDDLE), prd_w,
                         prd_b.reshape(1, DICT))
    agg = _sc_agg(x, ei, zeros)
    return _tc_mlp(x, agg, w, b)


# speedup vs baseline: 1.4768x; 1.0191x over previous
import functools

import jax
import jax.numpy as jnp
from jax import lax
from jax.experimental import pallas as pl
from jax.experimental.pallas import tpu as pltpu
from jax.experimental.pallas import tpu_sc as plsc

N_NODES = 10000
N_EDGES = 320000
HIDDEN = 128
MIDDLE = 320
DICT = 512

NUM_CORES = 2
NUM_SUBCORES = 16
NUM_WORKERS = NUM_CORES * NUM_SUBCORES
BATCH = 80
BPC = 25
CHUNKS = N_EDGES // (NUM_WORKERS * BPC * BATCH)
STRIPE = 624
TAIL = N_NODES - NUM_SUBCORES * STRIPE


@functools.partial(
    pl.kernel,
    mesh=plsc.VectorSubcoreMesh(core_axis_name="c", subcore_axis_name="s"),
    out_type=jax.ShapeDtypeStruct((NUM_CORES, N_NODES, HIDDEN), jnp.float32),
    scratch_types=[
        pltpu.VMEM((BPC, BATCH), jnp.int32),
        pltpu.VMEM((BPC, BATCH), jnp.int32),
        pltpu.VMEM((BATCH, HIDDEN), jnp.float32),
        pltpu.VMEM((BATCH, HIDDEN), jnp.float32),
        pltpu.VMEM((BATCH, HIDDEN), jnp.float32),
        pltpu.VMEM((BATCH, HIDDEN), jnp.float32),
        pltpu.VMEM_SHARED((N_NODES, HIDDEN), jnp.float32),
        pltpu.SemaphoreType.DMA,
        pltpu.SemaphoreType.DMA,
        pltpu.SemaphoreType.DMA,
        pltpu.SemaphoreType.DMA,
        pltpu.SemaphoreType.DMA,
        pltpu.SemaphoreType.DMA,
        pltpu.SemaphoreType.DMA,
        pltpu.SemaphoreType.DMA,
    ],
)
def _sc_agg(x_hbm, ei_hbm, zeros_hbm, out_hbm, src_v, dst_v,
            r0, r1, r2, r3, acc, g0, g1, g2, g3, s0, s1, s2, s3):
    c = lax.axis_index("c")
    s = lax.axis_index("s")
    wid = s * NUM_CORES + c

    @pl.when(c == 0)
    def _init_x():
        pltpu.sync_copy(x_hbm.at[pl.ds(s * STRIPE, STRIPE)],
                        acc.at[pl.ds(s * STRIPE, STRIPE)])

        @pl.when(s == 0)
        def _tail():
            pltpu.sync_copy(x_hbm.at[pl.ds(NUM_SUBCORES * STRIPE, TAIL)],
                            acc.at[pl.ds(NUM_SUBCORES * STRIPE, TAIL)])

    @pl.when(c == 1)
    def _init_zero():
        pltpu.sync_copy(zeros_hbm, acc.at[pl.ds(s * STRIPE, STRIPE)])

        @pl.when(s == 0)
        def _tail():
            pltpu.sync_copy(zeros_hbm.at[pl.ds(0, TAIL)],
                            acc.at[pl.ds(NUM_SUBCORES * STRIPE, TAIL)])

    plsc.subcore_barrier()

    rows = (r0, r1, r2, r3)
    gsem = (g0, g1, g2, g3)
    ssem = (s0, s1, s2, s3)

    def _wait_gather(u):
        pltpu.make_async_copy(x_hbm.at[src_v.at[0]], rows[u], gsem[u]).wait()

    def _wait_scatter(u):
        pltpu.make_async_copy(rows[u], acc.at[dst_v.at[0]], ssem[u]).wait()

    def _step(j, issue_j):
        u = j % 4
        _wait_gather(u)
        pltpu.async_copy(rows[u], acc.at[dst_v.at[j]], ssem[u], add=True)
        if issue_j is not None:
            v = issue_j % 4
            if issue_j >= 4:
                _wait_scatter(v)
            pltpu.async_copy(x_hbm.at[src_v.at[issue_j]], rows[v], gsem[v])

    def _chunk(k, carry):
        pltpu.sync_copy(ei_hbm.at[0, wid, k], src_v)
        pltpu.sync_copy(ei_hbm.at[1, wid, k], dst_v)
        for j in range(3):
            pltpu.async_copy(x_hbm.at[src_v.at[j]], rows[j], gsem[j])

        def _quad(i, cc):
            j0 = 4 * i

            def _dyn_step(u):
                _wait_gather(u)
                pltpu.async_copy(rows[u], acc.at[dst_v.at[j0 + u]], ssem[u],
                                 add=True)
                v = (u + 3) % 4
                @pl.when(j0 + u >= 1)
                def _():
                    _wait_scatter(v)
                pltpu.async_copy(x_hbm.at[src_v.at[j0 + u + 3]], rows[v],
                                 gsem[v])

            for u in range(4):
                _dyn_step(u)
            return cc

        lax.fori_loop(0, (BPC - 5) // 4, _quad, 0)
        _step(BPC - 5, BPC - 2)
        _step(BPC - 4, BPC - 1)
        _step(BPC - 3, None)
        _step(BPC - 2, None)
        _step(BPC - 1, None)
        for u in range(4):
            _wait_scatter(u)
        return carry

    lax.fori_loop(0, CHUNKS, _chunk, 0)
    plsc.subcore_barrier()

    pltpu.sync_copy(
        acc.at[pl.ds(s * STRIPE, STRIPE)],
        out_hbm.at[c, pl.ds(s * STRIPE, STRIPE)],
    )

    @pl.when(s == 0)
    def _write_tail():
        pltpu.sync_copy(
            acc.at[pl.ds(NUM_SUBCORES * STRIPE, TAIL)],
            out_hbm.at[c, pl.ds(NUM_SUBCORES * STRIPE, TAIL)],
        )


def _fuse_body(tw_ref, tb_ref, pw_ref, pb_ref, w_ref, b_ref):
    w_ref[...] = jnp.dot(tw_ref[...], pw_ref[...],
                         preferred_element_type=jnp.float32)
    b_ref[...] = jnp.dot(tb_ref[...], pw_ref[...],
                         preferred_element_type=jnp.float32) + pb_ref[...]


def _fuse_weights(trn_w, trn_b, prd_w, prd_b):
    return pl.pallas_call(
        _fuse_body,
        out_shape=(
            jax.ShapeDtypeStruct((HIDDEN, DICT), jnp.float32),
            jax.ShapeDtypeStruct((1, DICT), jnp.float32),
        ),
    )(trn_w, trn_b, prd_w, prd_b)


def _mlp_body(a_ref, w_ref, b_ref, o_ref):
    h = a_ref[0] + a_ref[1]
    o_ref[...] = jnp.dot(h, w_ref[...],
                         preferred_element_type=jnp.float32) + b_ref[...]


ROW_BLOCK = 1000


def _tc_mlp(agg, w, b):
    return pl.pallas_call(
        _mlp_body,
        grid=(N_NODES // ROW_BLOCK,),
        in_specs=[
            pl.BlockSpec((NUM_CORES, ROW_BLOCK, HIDDEN), lambda i: (0, i, 0)),
            pl.BlockSpec((HIDDEN, DICT), lambda i: (0, 0)),
            pl.BlockSpec((1, DICT), lambda i: (0, 0)),
        ],
        out_specs=pl.BlockSpec((ROW_BLOCK, DICT), lambda i: (i, 0)),
        out_shape=jax.ShapeDtypeStruct((N_NODES, DICT), jnp.float32),
    )(agg, w, b)


def kernel(x, edge_index, trn_w, trn_b, prd_w, prd_b):
    ei = edge_index.astype(jnp.int32).reshape(2, NUM_WORKERS, CHUNKS, BPC, BATCH)
    zeros = jnp.zeros((STRIPE, HIDDEN), jnp.float32)
    w, b = _fuse_weights(trn_w, trn_b.reshape(1, MIDDLE), prd_w,
                         prd_b.reshape(1, DICT))
    agg = _sc_agg(x, ei, zeros)
    return _tc_mlp(agg, w, b)


# speedup vs baseline: 1.5203x; 1.0294x over previous
import functools

import jax
import jax.numpy as jnp
from jax import lax
from jax.experimental import pallas as pl
from jax.experimental.pallas import tpu as pltpu
from jax.experimental.pallas import tpu_sc as plsc

N_NODES = 10000
N_EDGES = 320000
HIDDEN = 128
MIDDLE = 320
DICT = 512

NUM_CORES = 2
NUM_SUBCORES = 16
NUM_WORKERS = NUM_CORES * NUM_SUBCORES
BATCH = 80
BPC = 25
CHUNKS = N_EDGES // (NUM_WORKERS * BPC * BATCH)
STRIPE = 624
TAIL = N_NODES - NUM_SUBCORES * STRIPE


@functools.partial(
    pl.kernel,
    mesh=plsc.VectorSubcoreMesh(core_axis_name="c", subcore_axis_name="s"),
    out_type=jax.ShapeDtypeStruct((NUM_CORES, N_NODES, HIDDEN), jnp.float32),
    scratch_types=[
        pltpu.VMEM((BPC, BATCH), jnp.int32),
        pltpu.VMEM((BPC, BATCH), jnp.int32),
        pltpu.VMEM((BATCH, HIDDEN), jnp.float32),
        pltpu.VMEM((BATCH, HIDDEN), jnp.float32),
        pltpu.VMEM((BATCH, HIDDEN), jnp.float32),
        pltpu.VMEM((BATCH, HIDDEN), jnp.float32),
        pltpu.VMEM_SHARED((N_NODES, HIDDEN), jnp.float32),
        pltpu.SemaphoreType.DMA,
        pltpu.SemaphoreType.DMA,
        pltpu.SemaphoreType.DMA,
        pltpu.SemaphoreType.DMA,
        pltpu.SemaphoreType.DMA,
        pltpu.SemaphoreType.DMA,
        pltpu.SemaphoreType.DMA,
        pltpu.SemaphoreType.DMA,
        pltpu.SemaphoreType.DMA,
    ],
)
def _sc_agg(x_hbm, ei_hbm, zeros_hbm, out_hbm, src_v, dst_v,
            r0, r1, r2, r3, acc, g0, g1, g2, g3, s0, s1, s2, s3, stg):
    c = lax.axis_index("c")
    s = lax.axis_index("s")
    wid = s * NUM_CORES + c

    def _stage(k):
        pltpu.async_copy(ei_hbm.at[0, wid, k], src_v, stg)
        pltpu.async_copy(ei_hbm.at[1, wid, k], dst_v, stg)
        pltpu.make_async_copy(ei_hbm.at[0, wid, k], src_v, stg).wait()
        pltpu.make_async_copy(ei_hbm.at[1, wid, k], dst_v, stg).wait()

    pltpu.async_copy(ei_hbm.at[0, wid, 0], src_v, stg)
    pltpu.async_copy(ei_hbm.at[1, wid, 0], dst_v, stg)

    @pl.when(c == 0)
    def _init_x():
        pltpu.sync_copy(x_hbm.at[pl.ds(s * STRIPE, STRIPE)],
                        acc.at[pl.ds(s * STRIPE, STRIPE)])

        @pl.when(s == 0)
        def _tail():
            pltpu.sync_copy(x_hbm.at[pl.ds(NUM_SUBCORES * STRIPE, TAIL)],
                            acc.at[pl.ds(NUM_SUBCORES * STRIPE, TAIL)])

    @pl.when(c == 1)
    def _init_zero():
        pltpu.sync_copy(zeros_hbm, acc.at[pl.ds(s * STRIPE, STRIPE)])

        @pl.when(s == 0)
        def _tail():
            pltpu.sync_copy(zeros_hbm.at[pl.ds(0, TAIL)],
                            acc.at[pl.ds(NUM_SUBCORES * STRIPE, TAIL)])

    pltpu.make_async_copy(ei_hbm.at[0, wid, 0], src_v, stg).wait()
    pltpu.make_async_copy(ei_hbm.at[1, wid, 0], dst_v, stg).wait()

    plsc.subcore_barrier()

    rows = (r0, r1, r2, r3)
    gsem = (g0, g1, g2, g3)
    ssem = (s0, s1, s2, s3)

    def _wait_gather(u):
        pltpu.make_async_copy(x_hbm.at[src_v.at[0]], rows[u], gsem[u]).wait()

    def _wait_scatter(u):
        pltpu.make_async_copy(rows[u], acc.at[dst_v.at[0]], ssem[u]).wait()

    def _step(j, issue_j):
        u = j % 4
        _wait_gather(u)
        pltpu.async_copy(rows[u], acc.at[dst_v.at[j]], ssem[u], add=True)
        if issue_j is not None:
            v = issue_j % 4
            if issue_j >= 4:
                _wait_scatter(v)
            pltpu.async_copy(x_hbm.at[src_v.at[issue_j]], rows[v], gsem[v])

    for j in range(3):
        pltpu.async_copy(x_hbm.at[src_v.at[j]], rows[j], gsem[j])

    def _chunk(k, carry):
        def _quad(i, cc):
            j0 = 4 * i

            def _dyn_step(u):
                _wait_gather(u)
                pltpu.async_copy(rows[u], acc.at[dst_v.at[j0 + u]], ssem[u],
                                 add=True)
                v = (u + 3) % 4
                @pl.when(j0 + u >= 1)
                def _():
                    _wait_scatter(v)
                pltpu.async_copy(x_hbm.at[src_v.at[j0 + u + 3]], rows[v],
                                 gsem[v])

            for u in range(4):
                _dyn_step(u)
            return cc

        lax.fori_loop(0, (BPC - 5) // 4, _quad, 0)
        _step(BPC - 5, BPC - 2)
        _step(BPC - 4, BPC - 1)
        _step(BPC - 3, None)
        _step(BPC - 2, None)
        _step(BPC - 1, None)
        for u in range(4):
            _wait_scatter(u)

        @pl.when(k < CHUNKS - 1)
        def _next():
            _stage(k + 1)
            for j in range(3):
                pltpu.async_copy(x_hbm.at[src_v.at[j]], rows[j], gsem[j])

        return carry

    lax.fori_loop(0, CHUNKS, _chunk, 0)
    plsc.subcore_barrier()

    pltpu.sync_copy(
        acc.at[pl.ds(s * STRIPE, STRIPE)],
        out_hbm.at[c, pl.ds(s * STRIPE, STRIPE)],
    )

    @pl.when(s == 0)
    def _write_tail():
        pltpu.sync_copy(
            acc.at[pl.ds(NUM_SUBCORES * STRIPE, TAIL)],
            out_hbm.at[c, pl.ds(NUM_SUBCORES * STRIPE, TAIL)],
        )


def _fuse_body(tw_ref, tb_ref, pw_ref, pb_ref, w_ref, b_ref):
    w_ref[...] = jnp.dot(tw_ref[...], pw_ref[...],
                         preferred_element_type=jnp.float32)
    b_ref[...] = jnp.dot(tb_ref[...], pw_ref[...],
                         preferred_element_type=jnp.float32) + pb_ref[...]


def _fuse_weights(trn_w, trn_b, prd_w, prd_b):
    return pl.pallas_call(
        _fuse_body,
        out_shape=(
            jax.ShapeDtypeStruct((HIDDEN, DICT), jnp.float32),
            jax.ShapeDtypeStruct((1, DICT), jnp.float32),
        ),
    )(trn_w, trn_b, prd_w, prd_b)


def _mlp_body(a_ref, w_ref, b_ref, o_ref):
    h = a_ref[0] + a_ref[1]
    o_ref[...] = jnp.dot(h, w_ref[...],
                         preferred_element_type=jnp.float32) + b_ref[...]


ROW_BLOCK = 1000


def _tc_mlp(agg, w, b):
    return pl.pallas_call(
        _mlp_body,
        grid=(N_NODES // ROW_BLOCK,),
        in_specs=[
            pl.BlockSpec((NUM_CORES, ROW_BLOCK, HIDDEN), lambda i: (0, i, 0)),
            pl.BlockSpec((HIDDEN, DICT), lambda i: (0, 0)),
            pl.BlockSpec((1, DICT), lambda i: (0, 0)),
        ],
        out_specs=pl.BlockSpec((ROW_BLOCK, DICT), lambda i: (i, 0)),
        out_shape=jax.ShapeDtypeStruct((N_NODES, DICT), jnp.float32),
    )(agg, w, b)


def kernel(x, edge_index, trn_w, trn_b, prd_w, prd_b):
    ei = edge_index.astype(jnp.int32).reshape(2, NUM_WORKERS, CHUNKS, BPC, BATCH)
    zeros = jnp.zeros((STRIPE, HIDDEN), jnp.float32)
    w, b = _fuse_weights(trn_w, trn_b.reshape(1, MIDDLE), prd_w,
                         prd_b.reshape(1, DICT))
    agg = _sc_agg(x, ei, zeros)
    return _tc_mlp(agg, w, b)


# speedup vs baseline: 1.5583x; 1.0250x over previous
import functools

import jax
import jax.numpy as jnp
from jax import lax
from jax.experimental import pallas as pl
from jax.experimental.pallas import tpu as pltpu
from jax.experimental.pallas import tpu_sc as plsc

N_NODES = 10000
N_EDGES = 320000
HIDDEN = 128
MIDDLE = 320
DICT = 512

NUM_CORES = 2
NUM_SUBCORES = 16
NUM_WORKERS = NUM_CORES * NUM_SUBCORES
BATCH = 80
BPC = 25
CHUNKS = N_EDGES // (NUM_WORKERS * BPC * BATCH)
STRIPE = 624
TAIL = N_NODES - NUM_SUBCORES * STRIPE


@functools.partial(
    pl.kernel,
    mesh=plsc.VectorSubcoreMesh(core_axis_name="c", subcore_axis_name="s"),
    out_type=jax.ShapeDtypeStruct((NUM_CORES, N_NODES, HIDDEN), jnp.float32),
    scratch_types=[
        pltpu.VMEM((BPC, BATCH), jnp.int32),
        pltpu.VMEM((BPC, BATCH), jnp.int32),
        pltpu.VMEM((BATCH, HIDDEN), jnp.float32),
        pltpu.VMEM((BATCH, HIDDEN), jnp.float32),
        pltpu.VMEM((BATCH, HIDDEN), jnp.float32),
        pltpu.VMEM((BATCH, HIDDEN), jnp.float32),
        pltpu.VMEM_SHARED((N_NODES, HIDDEN), jnp.float32),
        pltpu.SemaphoreType.DMA,
        pltpu.SemaphoreType.DMA,
        pltpu.SemaphoreType.DMA,
        pltpu.SemaphoreType.DMA,
        pltpu.SemaphoreType.DMA,
        pltpu.SemaphoreType.DMA,
        pltpu.SemaphoreType.DMA,
        pltpu.SemaphoreType.DMA,
        pltpu.SemaphoreType.DMA,
    ],
)
def _sc_agg(x_hbm, ei_hbm, zeros_hbm, out_hbm, src_v, dst_v,
            r0, r1, r2, r3, acc, g0, g1, g2, g3, s0, s1, s2, s3, stg):
    c = lax.axis_index("c")
    s = lax.axis_index("s")
    wid = s * NUM_CORES + c
    rows = (r0, r1, r2, r3)
    gsem = (g0, g1, g2, g3)
    ssem = (s0, s1, s2, s3)

    def _stage(k):
        pltpu.async_copy(ei_hbm.at[0, wid, k], src_v, stg)
        pltpu.async_copy(ei_hbm.at[1, wid, k], dst_v, stg)
        pltpu.make_async_copy(ei_hbm.at[0, wid, k], src_v, stg).wait()
        pltpu.make_async_copy(ei_hbm.at[1, wid, k], dst_v, stg).wait()

    pltpu.async_copy(ei_hbm.at[0, wid, 0], src_v, stg)
    pltpu.async_copy(ei_hbm.at[1, wid, 0], dst_v, stg)

    @pl.when(c == 0)
    def _init_x():
        pltpu.sync_copy(x_hbm.at[pl.ds(s * STRIPE, STRIPE)],
                        acc.at[pl.ds(s * STRIPE, STRIPE)])

        @pl.when(s == 0)
        def _tail():
            pltpu.sync_copy(x_hbm.at[pl.ds(NUM_SUBCORES * STRIPE, TAIL)],
                            acc.at[pl.ds(NUM_SUBCORES * STRIPE, TAIL)])

    @pl.when(c == 1)
    def _init_zero():
        pltpu.sync_copy(zeros_hbm, acc.at[pl.ds(s * STRIPE, STRIPE)])

        @pl.when(s == 0)
        def _tail():
            pltpu.sync_copy(zeros_hbm.at[pl.ds(0, TAIL)],
                            acc.at[pl.ds(NUM_SUBCORES * STRIPE, TAIL)])

    pltpu.make_async_copy(ei_hbm.at[0, wid, 0], src_v, stg).wait()
    pltpu.make_async_copy(ei_hbm.at[1, wid, 0], dst_v, stg).wait()
    for j in range(3):
        pltpu.async_copy(x_hbm.at[src_v.at[j]], rows[j], gsem[j])

    plsc.subcore_barrier()

    def _wait_gather(u):
        pltpu.make_async_copy(x_hbm.at[src_v.at[0]], rows[u], gsem[u]).wait()

    def _wait_scatter(u):
        pltpu.make_async_copy(rows[u], acc.at[dst_v.at[0]], ssem[u]).wait()

    def _step(j, issue_j):
        u = j % 4
        _wait_gather(u)
        pltpu.async_copy(rows[u], acc.at[dst_v.at[j]], ssem[u], add=True)
        if issue_j is not None:
            v = issue_j % 4
            if issue_j >= 4:
                _wait_scatter(v)
            pltpu.async_copy(x_hbm.at[src_v.at[issue_j]], rows[v], gsem[v])

    def _chunk(k, carry):
        def _quad(i, cc):
            j0 = 4 * i

            def _dyn_step(u):
                _wait_gather(u)
                pltpu.async_copy(rows[u], acc.at[dst_v.at[j0 + u]], ssem[u],
                                 add=True)
                v = (u + 3) % 4
                @pl.when(j0 + u >= 1)
                def _():
                    _wait_scatter(v)
                pltpu.async_copy(x_hbm.at[src_v.at[j0 + u + 3]], rows[v],
                                 gsem[v])

            for u in range(4):
                _dyn_step(u)
            return cc

        lax.fori_loop(0, (BPC - 5) // 4, _quad, 0)
        _step(BPC - 5, BPC - 2)
        _step(BPC - 4, BPC - 1)
        _step(BPC - 3, None)
        _step(BPC - 2, None)
        _step(BPC - 1, None)
        for u in range(4):
            _wait_scatter(u)

        @pl.when(k < CHUNKS - 1)
        def _next():
            _stage(k + 1)
            for j in range(3):
                pltpu.async_copy(x_hbm.at[src_v.at[j]], rows[j], gsem[j])

        return carry

    lax.fori_loop(0, CHUNKS, _chunk, 0)
    plsc.subcore_barrier()

    pltpu.sync_copy(
        acc.at[pl.ds(s * STRIPE, STRIPE)],
        out_hbm.at[c, pl.ds(s * STRIPE, STRIPE)],
    )

    @pl.when(s == 0)
    def _write_tail():
        pltpu.sync_copy(
            acc.at[pl.ds(NUM_SUBCORES * STRIPE, TAIL)],
            out_hbm.at[c, pl.ds(NUM_SUBCORES * STRIPE, TAIL)],
        )


def _fuse_body(tw_ref, tb_ref, pw_ref, pb_ref, w_ref, b_ref):
    w_ref[...] = jnp.dot(tw_ref[...], pw_ref[...],
                         preferred_element_type=jnp.float32)
    b_ref[...] = jnp.dot(tb_ref[...], pw_ref[...],
                         preferred_element_type=jnp.float32) + pb_ref[...]


def _fuse_weights(trn_w, trn_b, prd_w, prd_b):
    return pl.pallas_call(
        _fuse_body,
        out_shape=(
            jax.ShapeDtypeStruct((HIDDEN, DICT), jnp.float32),
            jax.ShapeDtypeStruct((1, DICT), jnp.float32),
        ),
    )(trn_w, trn_b, prd_w, prd_b)


def _mlp_body(a_ref, w_ref, b_ref, o_ref):
    h = a_ref[0] + a_ref[1]
    o_ref[...] = jnp.dot(h, w_ref[...],
                         preferred_element_type=jnp.float32) + b_ref[...]


ROW_BLOCK = 2000


def _tc_mlp(agg, w, b):
    return pl.pallas_call(
        _mlp_body,
        grid=(N_NODES // ROW_BLOCK,),
        in_specs=[
            pl.BlockSpec((NUM_CORES, ROW_BLOCK, HIDDEN), lambda i: (0, i, 0)),
            pl.BlockSpec((HIDDEN, DICT), lambda i: (0, 0)),
            pl.BlockSpec((1, DICT), lambda i: (0, 0)),
        ],
        out_specs=pl.BlockSpec((ROW_BLOCK, DICT), lambda i: (i, 0)),
        out_shape=jax.ShapeDtypeStruct((N_NODES, DICT), jnp.float32),
    )(agg, w, b)


def kernel(x, edge_index, trn_w, trn_b, prd_w, prd_b):
    ei = edge_index.astype(jnp.int32).reshape(2, NUM_WORKERS, CHUNKS, BPC, BATCH)
    zeros = jnp.zeros((STRIPE, HIDDEN), jnp.float32)
    w, b = _fuse_weights(trn_w, trn_b.reshape(1, MIDDLE), prd_w,
                         prd_b.reshape(1, DICT))
    agg = _sc_agg(x, ei, zeros)
    return _tc_mlp(agg, w, b)
